# Initial kernel scaffold; baseline (speedup 1.0000x reference)
import jax
import jax.numpy as jnp
from jax.experimental import pallas as pl


def kernel(features, edge_index, edge_weight, kernel, bias, skip_weight):
    raise NotImplementedError("write your pallas kernel here")


# speedup vs baseline: 4.5841x; 4.5841x over previous
import functools

import jax
import jax.numpy as jnp
from jax import lax
from jax.experimental import pallas as pl
from jax.experimental.pallas import tpu as pltpu
from jax.experimental.pallas import tpu_sc as plsc

NC = 2
NS = 16
NW = NC * NS
L = 16
CHUNK = 128


def _mm_body(x_ref, w_ref, o_ref):
    o_ref[...] = jnp.dot(x_ref[...], w_ref[...],
                         preferred_element_type=jnp.float32)


def _matmul(x, w):
    n, d_in = x.shape
    d_out = w.shape[1]
    bm = 2000
    grid = (n // bm,)
    return pl.pallas_call(
        _mm_body,
        grid=grid,
        in_specs=[
            pl.BlockSpec((bm, d_in), lambda i: (i, 0)),
            pl.BlockSpec((d_in, d_out), lambda i: (0, 0)),
        ],
        out_specs=pl.BlockSpec((bm, d_out), lambda i: (i, 0)),
        out_shape=jax.ShapeDtypeStruct((n, d_out), jnp.float32),
    )(x, w)


def _sc_agg_body(nch, n, xw_hbm, src_hbm, dst_hbm, w_hbm, part_hbm,
                 src_v, dst_v, w_v, rows_v, acc_sh, sem):
    c = lax.axis_index("c")
    s = lax.axis_index("s")
    wid = s * NC + c

    pltpu.sync_copy(src_hbm.at[wid], src_v)
    pltpu.sync_copy(dst_hbm.at[wid], dst_v)
    pltpu.sync_copy(w_hbm.at[wid], w_v)

    zero = jnp.zeros((L,), jnp.float32)

    def _zb(i, _):
        rows_v[i // 8, pl.ds((i % 8) * L, L)] = zero
        return 0

    lax.fori_loop(0, CHUNK * 8, _zb, 0)

    rpt = (n // (8 * NS)) * 8
    tail = n - NS * rpt
    base = s * rpt
    nfull = rpt // CHUNK
    rem = rpt - nfull * CHUNK

    def _zacc(i, _):
        pltpu.sync_copy(rows_v, acc_sh.at[pl.ds(base + i * CHUNK, CHUNK)])
        return 0

    lax.fori_loop(0, nfull, _zacc, 0)
    if rem:
        pltpu.sync_copy(rows_v.at[pl.ds(0, rem)],
                        acc_sh.at[pl.ds(base + nfull * CHUNK, rem)])
    if tail:
        @pl.when(s == NS - 1)
        def _ztail():
            pltpu.sync_copy(rows_v.at[pl.ds(0, tail)],
                            acc_sh.at[pl.ds(NS * rpt, tail)])

    plsc.subcore_barrier()

    def _chunk(j, _):
        pltpu.async_copy(xw_hbm.at[src_v.at[j]], rows_v, sem).wait()

        joff = j * CHUNK

        def _scale(g, _):
            wvec = w_v[pl.ds(joff + g * L, L)]
            ebase = g * L
            for i in range(L):
                wv = wvec[i]
                for cg in range(8):
                    sl = pl.ds(cg * L, L)
                    rows_v[ebase + i, sl] = rows_v[ebase + i, sl] * wv
            return 0

        lax.fori_loop(0, CHUNK // L, _scale, 0)

        pltpu.sync_copy(rows_v, acc_sh.at[dst_v.at[j]], add=True)
        return 0

    lax.fori_loop(0, nch, _chunk, 0)
    plsc.subcore_barrier()

    rpt = (n // (8 * NS)) * 8
    tail = n - NS * rpt
    base = s * rpt
    pltpu.sync_copy(acc_sh.at[pl.ds(base, rpt)],
                    part_hbm.at[c, pl.ds(base, rpt)])
    if tail:
        @pl.when(s == NS - 1)
        def _wtail():
            pltpu.sync_copy(acc_sh.at[pl.ds(NS * rpt, tail)],
                            part_hbm.at[c, pl.ds(NS * rpt, tail)])


def _sc_aggregate(xw, src, dst, ew):
    n, d = xw.shape
    e = src.shape[0]
    step = NW * CHUNK
    e_pad = ((e + step - 1) // step) * step
    pad = e_pad - e
    if pad:
        src = jnp.concatenate([src, jnp.zeros((pad,), jnp.int32)])
        dst = jnp.concatenate([dst, jnp.zeros((pad,), jnp.int32)])
        ew = jnp.concatenate([ew, jnp.zeros((pad,), jnp.float32)])
    nch = e_pad // step
    src = src.reshape(NW, nch, CHUNK)
    dst = dst.reshape(NW, nch, CHUNK)
    ew = ew.reshape(NW, nch * CHUNK)

    mesh = plsc.VectorSubcoreMesh(core_axis_name="c", subcore_axis_name="s")
    k = functools.partial(
        pl.kernel,
        mesh=mesh,
        out_type=jax.ShapeDtypeStruct((NC, n, d), jnp.float32),
        scratch_types=[
            pltpu.VMEM((nch, CHUNK), jnp.int32),
            pltpu.VMEM((nch, CHUNK), jnp.int32),
            pltpu.VMEM((nch * CHUNK,), jnp.float32),
            pltpu.VMEM((CHUNK, d), jnp.float32),
            pltpu.VMEM_SHARED((n, d), jnp.float32),
            pltpu.SemaphoreType.DMA,
        ],
    )(functools.partial(_sc_agg_body, nch, n))
    return k(xw, src, dst, ew)


def _fin_body(xw_ref, p_ref, skip_ref, bias_ref, o_ref):
    v = (xw_ref[...] * skip_ref[...] + p_ref[0] + p_ref[1] + bias_ref[...])
    alpha = 1.6732632423543772848170429916717
    scale = 1.0507009873554804934193349852946
    o_ref[...] = scale * jnp.where(v > 0, v, alpha * (jnp.exp(v) - 1.0))


def _finalize(xw, parts, skip_weight, bias):
    n, d = xw.shape
    bm = 2000
    grid = (n // bm,)
    return pl.pallas_call(
        _fin_body,
        grid=grid,
        in_specs=[
            pl.BlockSpec((bm, d), lambda i: (i, 0)),
            pl.BlockSpec((NC, bm, d), lambda i: (0, i, 0)),
            pl.BlockSpec((1, d), lambda i: (0, 0)),
            pl.BlockSpec((1, d), lambda i: (0, 0)),
        ],
        out_specs=pl.BlockSpec((bm, d), lambda i: (i, 0)),
        out_shape=jax.ShapeDtypeStruct((n, d), jnp.float32),
    )(xw, parts, skip_weight.reshape(1, d), bias.reshape(1, d))


def kernel(features, edge_index, edge_weight, kernel, bias, skip_weight):
    xw = _matmul(features, kernel)
    parts = _sc_aggregate(xw, edge_index[0], edge_index[1], edge_weight)
    return _finalize(xw, parts, skip_weight, bias)
